# Initial kernel scaffold; baseline (speedup 1.0000x reference)
import jax
import jax.numpy as jnp
from jax.experimental import pallas as pl


def kernel(inputs, weight):
    raise NotImplementedError("write your pallas kernel here")


# speedup vs baseline: 1.8908x; 1.8908x over previous
import jax
import jax.numpy as jnp
from jax.experimental import pallas as pl

_NUM_K = 1024
_DIM = 64
_TILE = 512
_COMMITMENT_COST = 0.25


def _vq_body(x_ref, xsq_ref, w_ref, wsq_ref, q_ref, loss_ref):
    x = x_ref[...]
    w = w_ref[...]
    mm = jax.lax.dot_general(
        x, w, (((1,), (1,)), ((), ())),
        preferred_element_type=jnp.float32)
    dist = (xsq_ref[...] + wsq_ref[...]) - 2.0 * mm
    minval = jnp.min(dist, axis=1, keepdims=True)
    kiota = jax.lax.broadcasted_iota(jnp.int32, (_TILE, _NUM_K), 1)
    idx = jnp.min(jnp.where(dist == minval, kiota, _NUM_K),
                  axis=1, keepdims=True)
    enc = (kiota == idx).astype(jnp.float32)
    q = jax.lax.dot_general(
        enc, w, (((1,), (0,)), ((), ())),
        preferred_element_type=jnp.float32)
    q_ref[...] = x + (q - x)
    part = jnp.sum((q - x) ** 2)

    @pl.when(pl.program_id(0) == 0)
    def _init():
        loss_ref[...] = jnp.zeros_like(loss_ref)

    loss_ref[...] = loss_ref[...] + part


def kernel(inputs, weight):
    b, c, h, w_sz = inputs.shape
    x = jnp.transpose(inputs, (0, 2, 3, 1)).reshape(-1, _DIM)
    n = x.shape[0]
    xsq = jnp.sum(x ** 2, axis=1, keepdims=True)
    wsq = jnp.sum(weight ** 2, axis=1)[None, :]

    q, loss_acc = pl.pallas_call(
        _vq_body,
        grid=(n // _TILE,),
        in_specs=[
            pl.BlockSpec((_TILE, _DIM), lambda i: (i, 0)),
            pl.BlockSpec((_TILE, 1), lambda i: (i, 0)),
            pl.BlockSpec((_NUM_K, _DIM), lambda i: (0, 0)),
            pl.BlockSpec((1, _NUM_K), lambda i: (0, 0)),
        ],
        out_specs=[
            pl.BlockSpec((_TILE, _DIM), lambda i: (i, 0)),
            pl.BlockSpec((1, 1), lambda i: (0, 0)),
        ],
        out_shape=[
            jax.ShapeDtypeStruct((n, _DIM), jnp.float32),
            jax.ShapeDtypeStruct((1, 1), jnp.float32),
        ],
    )(x, xsq, weight, wsq)

    mean_sq = loss_acc[0, 0] / (n * _DIM)
    loss = mean_sq + _COMMITMENT_COST * mean_sq
    quantized_out = jnp.transpose(q.reshape(b, h, w_sz, c), (0, 3, 1, 2))
    return (quantized_out, loss)


# speedup vs baseline: 2.3076x; 1.2204x over previous
import jax
import jax.numpy as jnp
from jax.experimental import pallas as pl

_NUM_K = 1024
_DIM = 64
_TILE = 512
_COMMITMENT_COST = 0.25


def _vq_body(x_ref, xsq_ref, w_ref, wsq_ref, q_ref, loss_ref):
    x = x_ref[0]
    w = w_ref[...]
    mm = jax.lax.dot_general(
        w, x, (((1,), (0,)), ((), ())),
        preferred_element_type=jnp.float32)
    dist = (wsq_ref[...] + xsq_ref[0]) - 2.0 * mm
    minval = jnp.min(dist, axis=0, keepdims=True)
    kiota = jax.lax.broadcasted_iota(jnp.int32, (_NUM_K, _TILE), 0)
    idx = jnp.min(jnp.where(dist == minval, kiota, _NUM_K),
                  axis=0, keepdims=True)
    enc = (kiota == idx).astype(jnp.float32)
    q = jax.lax.dot_general(
        w, enc, (((0,), (0,)), ((), ())),
        preferred_element_type=jnp.float32)
    q_ref[0] = x + (q - x)
    part = jnp.sum(minval)

    @pl.when((pl.program_id(0) == 0) & (pl.program_id(1) == 0))
    def _init():
        loss_ref[...] = jnp.zeros_like(loss_ref)

    loss_ref[...] = loss_ref[...] + part


def kernel(inputs, weight):
    b, c, h, w_sz = inputs.shape
    hw = h * w_sz
    x = inputs.reshape(b, c, hw)
    xsq = jnp.sum(x ** 2, axis=1, keepdims=True)
    wsq = jnp.sum(weight ** 2, axis=1)[:, None]

    q, loss_acc = pl.pallas_call(
        _vq_body,
        grid=(b, hw // _TILE),
        in_specs=[
            pl.BlockSpec((1, _DIM, _TILE), lambda i, j: (i, 0, j)),
            pl.BlockSpec((1, 1, _TILE), lambda i, j: (i, 0, j)),
            pl.BlockSpec((_NUM_K, _DIM), lambda i, j: (0, 0)),
            pl.BlockSpec((_NUM_K, 1), lambda i, j: (0, 0)),
        ],
        out_specs=[
            pl.BlockSpec((1, _DIM, _TILE), lambda i, j: (i, 0, j)),
            pl.BlockSpec((1, 1), lambda i, j: (0, 0)),
        ],
        out_shape=[
            jax.ShapeDtypeStruct((b, _DIM, hw), jnp.float32),
            jax.ShapeDtypeStruct((1, 1), jnp.float32),
        ],
    )(x, xsq, weight, wsq)

    n_total = b * hw * _DIM
    mean_sq = loss_acc[0, 0] / n_total
    loss = mean_sq + _COMMITMENT_COST * mean_sq
    return (q.reshape(b, c, h, w_sz), loss)


# speedup vs baseline: 2.5231x; 1.0934x over previous
import jax
import jax.numpy as jnp
from jax.experimental import pallas as pl

_NUM_K = 1024
_DIM = 64
_TILE = 1024
_COMMITMENT_COST = 0.25


def _vq_body(x_ref, xsq_ref, w_ref, w2n_ref, wsq_ref, kiota_ref, q_ref,
             loss_ref):
    x = x_ref[0]
    mm2 = jax.lax.dot_general(
        w2n_ref[...], x, (((1,), (0,)), ((), ())),
        preferred_element_type=jnp.float32)
    dist = (wsq_ref[...] + xsq_ref[0]) + mm2
    minval = jnp.min(dist, axis=0, keepdims=True)
    kiota = kiota_ref[...]
    idxf = jnp.min(jnp.where(dist == minval, kiota, float(_NUM_K)),
                   axis=0, keepdims=True)
    enc = (kiota == idxf).astype(jnp.float32)
    q = jax.lax.dot_general(
        w_ref[...], enc, (((0,), (0,)), ((), ())),
        preferred_element_type=jnp.float32)
    q_ref[0] = x + (q - x)
    part = jnp.sum(minval)

    @pl.when((pl.program_id(0) == 0) & (pl.program_id(1) == 0))
    def _init():
        loss_ref[...] = jnp.zeros_like(loss_ref)

    loss_ref[...] = loss_ref[...] + part


def kernel(inputs, weight):
    b, c, h, w_sz = inputs.shape
    hw = h * w_sz
    x = inputs.reshape(b, c, hw)
    xsq = jnp.sum(x ** 2, axis=1, keepdims=True)
    wsq = jnp.sum(weight ** 2, axis=1)[:, None]
    w2n = -2.0 * weight
    kiota = jax.lax.broadcasted_iota(jnp.float32, (_NUM_K, _TILE), 0)

    q, loss_acc = pl.pallas_call(
        _vq_body,
        grid=(b, hw // _TILE),
        in_specs=[
            pl.BlockSpec((1, _DIM, _TILE), lambda i, j: (i, 0, j)),
            pl.BlockSpec((1, 1, _TILE), lambda i, j: (i, 0, j)),
            pl.BlockSpec((_NUM_K, _DIM), lambda i, j: (0, 0)),
            pl.BlockSpec((_NUM_K, _DIM), lambda i, j: (0, 0)),
            pl.BlockSpec((_NUM_K, 1), lambda i, j: (0, 0)),
            pl.BlockSpec((_NUM_K, _TILE), lambda i, j: (0, 0)),
        ],
        out_specs=[
            pl.BlockSpec((1, _DIM, _TILE), lambda i, j: (i, 0, j)),
            pl.BlockSpec((1, 1), lambda i, j: (0, 0)),
        ],
        out_shape=[
            jax.ShapeDtypeStruct((b, _DIM, hw), jnp.float32),
            jax.ShapeDtypeStruct((1, 1), jnp.float32),
        ],
    )(x, xsq, weight, w2n, wsq, kiota)

    n_total = b * hw * _DIM
    mean_sq = loss_acc[0, 0] / n_total
    loss = mean_sq + _COMMITMENT_COST * mean_sq
    return (q.reshape(b, c, h, w_sz), loss)
